# baseline (speedup 1.0000x reference)
import functools

import jax
import jax.numpy as jnp
from jax import lax
from jax.experimental import pallas as pl
from jax.experimental.pallas import tpu as pltpu
from jax.experimental.pallas import tpu_sc as plsc

B, H, S, D, K = 64, 50, 32, 256, 8
THRESHOLD = 0.2
NEG = -1.0e30

NROWS = B * H * K
NC, NS = 2, 16
NW = NC * NS
PER_W = NROWS // NW
CHUNK = 160
BB = 4


def _tc_body(nse_ref, nr_ref, ur_ref, hm_ref, wt_ref, b_ref,
             kid_ref, w_ref, m_ref, g_ref):
    for bb in range(BB):
        _tc_one(bb, nse_ref, nr_ref, ur_ref, hm_ref, wt_ref, b_ref,
                kid_ref, w_ref, m_ref, g_ref)


def _tc_one(bb, nse_ref, nr_ref, ur_ref, hm_ref, wt_ref, b_ref,
            kid_ref, w_ref, m_ref, g_ref):
    bi = pl.program_id(0) * BB + bb
    nse = nse_ref[bb]
    nr = nr_ref[bb]
    u = ur_ref[bb]
    wm = wt_ref[...]

    nur = jnp.concatenate([jnp.broadcast_to(u, (H, D)), nr], axis=-1)
    q = lax.dot_general(nur, wm, (((1,), (1,)), ((), ())),
                        preferred_element_type=jnp.float32) + b_ref[...]
    qs = q / jnp.maximum(jnp.abs(q), 1e-12)

    ss = jnp.sum(nse * nse, axis=-1)
    rn = 1.0 / jnp.maximum(jnp.sqrt(ss), 1e-12)
    nsen = nse * rn[:, :, None]
    nb = nsen.astype(jnp.bfloat16).astype(jnp.float32)
    qb = qs.astype(jnp.bfloat16).astype(jnp.float32)
    sc = jnp.sum(nb * qb[:, None, :], axis=-1)

    s_iota = lax.broadcasted_iota(jnp.int32, (H, S), 1)
    sc = jnp.where(s_iota == 0, NEG, sc)
    hm = hm_ref[bb]

    rank = jnp.zeros((H, S), jnp.int32)
    for o in range(1, S):
        rolled = jnp.concatenate([sc[:, o:], sc[:, :o]], axis=-1)
        tie = (rolled == sc) & (s_iota >= S - o)
        rank = rank + ((rolled > sc) | tie).astype(jnp.int32)

    combo = s_iota + (hm << 5)
    ks, kcs = [], []
    for k in range(K):
        selk = rank == k
        ks.append(jnp.sum(jnp.where(selk, sc, 0.0), axis=-1, keepdims=True))
        kcs.append(jnp.sum(jnp.where(selk, combo, 0), axis=-1, keepdims=True))

    score_k = jnp.concatenate(ks, axis=-1)
    comb = jnp.concatenate(kcs, axis=-1)
    sidx = comb & 31
    mk = comb >> 5

    keep = score_k >= THRESHOLD
    w = jnp.where(keep, score_k, 0.0)
    mk = mk * keep.astype(jnp.int32)
    h_iota = lax.broadcasted_iota(jnp.int32, (H, K), 0)
    g = (bi * H + h_iota) * S + sidx

    kid_ref[bb] = sidx - 1
    w_ref[bb] = w
    m_ref[bb] = mk
    g_ref[bb] = g


def _tc_score_topk(nse, news_repr, user_repr, his_mask, wt, b2):
    nb_ = nse.shape[0]
    return pl.pallas_call(
        _tc_body,
        grid=(nb_ // BB,),
        in_specs=[
            pl.BlockSpec((BB, H, S, D), lambda i: (i, 0, 0, 0)),
            pl.BlockSpec((BB, H, D), lambda i: (i, 0, 0)),
            pl.BlockSpec((BB, 1, D), lambda i: (i, 0, 0)),
            pl.BlockSpec((BB, H, S), lambda i: (i, 0, 0)),
            pl.BlockSpec((D, 2 * D), lambda i: (0, 0)),
            pl.BlockSpec((1, D), lambda i: (0, 0)),
        ],
        out_specs=[
            pl.BlockSpec((BB, H, K), lambda i: (i, 0, 0)),
            pl.BlockSpec((BB, H, K), lambda i: (i, 0, 0)),
            pl.BlockSpec((BB, H, K), lambda i: (i, 0, 0)),
            pl.BlockSpec((BB, H, K), lambda i: (i, 0, 0)),
        ],
        out_shape=[
            jax.ShapeDtypeStruct((nb_, H, K), jnp.int32),
            jax.ShapeDtypeStruct((nb_, H, K), jnp.float32),
            jax.ShapeDtypeStruct((nb_, H, K), jnp.int32),
            jax.ShapeDtypeStruct((nb_, H, K), jnp.int32),
        ],
    )(nse, news_repr, user_repr, his_mask, wt, b2)


def _sc_body(per_w, chunk, table_hbm, idx_hbm, w_hbm, out_hbm, idx_v, w_v,
             rows0, rows1, g0, g1, o0, o1):
    wid = lax.axis_index("s") * NC + lax.axis_index("c")
    base = wid * per_w
    icp = pltpu.make_async_copy(idx_hbm.at[pl.ds(base, per_w)], idx_v, g0)
    wcp = pltpu.make_async_copy(w_hbm.at[pl.ds(base, per_w)], w_v, g1)
    icp.start()
    wcp.start()
    icp.wait()
    wcp.wait()

    nch = per_w // chunk
    bufs, gsems, osems = (rows0, rows1), (g0, g1), (o0, o1)
    gathers = [
        pltpu.make_async_copy(
            table_hbm.at[idx_v.at[pl.ds(c * chunk, chunk)]],
            bufs[c % 2], gsems[c % 2])
        for c in range(nch)
    ]
    outs = [
        pltpu.make_async_copy(
            bufs[c % 2], out_hbm.at[pl.ds(base + c * chunk, chunk)],
            osems[c % 2])
        for c in range(nch)
    ]

    def scale(buf, c):
        def group(gi, gcarry):
            w16 = w_v[pl.ds(c * chunk + gi * 16, 16)]
            for i in range(16):
                r = gi * 16 + i
                wvec = jnp.full((16,), w16[i], jnp.float32)
                for j in range(D // 16):
                    sl = pl.ds(j * 16, 16)
                    buf[r, sl] = buf[r, sl] * wvec
            return gcarry
        lax.fori_loop(0, chunk // 16, group, 0)

    gathers[0].start()
    for c in range(nch):
        if c + 1 < nch:
            if c >= 1:
                outs[c - 1].wait()
            gathers[c + 1].start()
        gathers[c].wait()
        scale(bufs[c % 2], c)
        outs[c].start()
    if nch >= 2:
        outs[nch - 2].wait()
    outs[nch - 1].wait()


@functools.lru_cache(maxsize=4)
def _sc_gather_scale(nrows, chunk):
    per_w = nrows // NW
    return pl.kernel(
        functools.partial(_sc_body, per_w, chunk),
        out_type=jax.ShapeDtypeStruct((nrows, D), jnp.float32),
        mesh=plsc.VectorSubcoreMesh(core_axis_name="c", subcore_axis_name="s"),
        scratch_types=[
            pltpu.VMEM((per_w,), jnp.int32),
            pltpu.VMEM((per_w,), jnp.float32),
            pltpu.VMEM((chunk, D), jnp.float32),
            pltpu.VMEM((chunk, D), jnp.float32),
            pltpu.SemaphoreType.DMA,
            pltpu.SemaphoreType.DMA,
            pltpu.SemaphoreType.DMA,
            pltpu.SemaphoreType.DMA,
        ],
    )


def kernel(news_selection_embedding, news_embedding, user_repr, news_repr,
           his_attn_mask, W, b):
    b2 = b.reshape(1, D)
    bh = B // 2
    nrows_h = bh * H * K
    halves = []
    for i in range(2):
        sl = slice(i * bh, (i + 1) * bh)
        kid_i, w_i, m_i, g_i = _tc_score_topk(
            news_selection_embedding[sl], news_repr[sl], user_repr[sl],
            his_attn_mask[sl], W, b2)
        table_i = news_embedding[sl].reshape(bh * H * S, D)
        ps_i = _sc_gather_scale(nrows_h, 80)(
            table_i, g_i.reshape(nrows_h), w_i.reshape(nrows_h))
        halves.append((ps_i, m_i, kid_i))
    ps = jnp.concatenate([halves[0][0], halves[1][0]], axis=0)
    mask = jnp.concatenate([halves[0][1], halves[1][1]], axis=0)
    kid = jnp.concatenate([halves[0][2], halves[1][2]], axis=0)
    return ps.reshape(B, H, K, D), mask, kid


# speedup vs baseline: 1.9705x; 1.9705x over previous
import functools

import jax
import jax.numpy as jnp
from jax import lax
from jax.experimental import pallas as pl
from jax.experimental.pallas import tpu as pltpu
from jax.experimental.pallas import tpu_sc as plsc

B, H, S, D, K = 64, 50, 32, 256, 8
THRESHOLD = 0.2
NEG = -1.0e30

NROWS = B * H * K
NC, NS = 2, 16
NW = NC * NS
PER_W = NROWS // NW
CHUNK = 160
BB = 4


def _tc_body(nse_ref, nr_ref, ur_ref, hm_ref, wt_ref, b_ref,
             kid_ref, w_ref, m_ref, g_ref):
    for bb in range(BB):
        _tc_one(bb, nse_ref, nr_ref, ur_ref, hm_ref, wt_ref, b_ref,
                kid_ref, w_ref, m_ref, g_ref)


def _tc_one(bb, nse_ref, nr_ref, ur_ref, hm_ref, wt_ref, b_ref,
            kid_ref, w_ref, m_ref, g_ref):
    bi = pl.program_id(0) * BB + bb
    nse = nse_ref[bb]
    nr = nr_ref[bb]
    u = ur_ref[bb]
    wm = wt_ref[...]

    nur = jnp.concatenate([jnp.broadcast_to(u, (H, D)), nr], axis=-1)
    q = lax.dot_general(nur, wm, (((1,), (1,)), ((), ())),
                        preferred_element_type=jnp.float32) + b_ref[...]
    qs = q / jnp.maximum(jnp.abs(q), 1e-12)

    ss = jnp.sum(nse * nse, axis=-1)
    rn = 1.0 / jnp.maximum(jnp.sqrt(ss), 1e-12)
    nsen = nse * rn[:, :, None]
    nb = nsen.astype(jnp.bfloat16).astype(jnp.float32)
    qb = qs.astype(jnp.bfloat16).astype(jnp.float32)
    sc = jnp.sum(nb * qb[:, None, :], axis=-1)

    s_iota = lax.broadcasted_iota(jnp.int32, (H, S), 1)
    sc = jnp.where(s_iota == 0, NEG, sc)
    hm = hm_ref[bb]

    rank = jnp.zeros((H, S), jnp.int32)
    for o in range(1, S):
        rolled = jnp.concatenate([sc[:, o:], sc[:, :o]], axis=-1)
        tie = (rolled == sc) & (s_iota >= S - o)
        rank = rank + ((rolled > sc) | tie).astype(jnp.int32)

    combo = s_iota + (hm << 5)
    ks, kcs = [], []
    for k in range(K):
        selk = rank == k
        ks.append(jnp.sum(jnp.where(selk, sc, 0.0), axis=-1, keepdims=True))
        kcs.append(jnp.sum(jnp.where(selk, combo, 0), axis=-1, keepdims=True))

    score_k = jnp.concatenate(ks, axis=-1)
    comb = jnp.concatenate(kcs, axis=-1)
    sidx = comb & 31
    mk = comb >> 5

    keep = score_k >= THRESHOLD
    w = jnp.where(keep, score_k, 0.0)
    mk = mk * keep.astype(jnp.int32)
    h_iota = lax.broadcasted_iota(jnp.int32, (H, K), 0)
    g = (bi * H + h_iota) * S + sidx

    kid_ref[bb] = sidx - 1
    w_ref[bb] = w
    m_ref[bb] = mk
    g_ref[bb] = g


def _tc_score_topk(nse, news_repr, user_repr, his_mask, wt, b2):
    nb_ = nse.shape[0]
    return pl.pallas_call(
        _tc_body,
        grid=(nb_ // BB,),
        in_specs=[
            pl.BlockSpec((BB, H, S, D), lambda i: (i, 0, 0, 0)),
            pl.BlockSpec((BB, H, D), lambda i: (i, 0, 0)),
            pl.BlockSpec((BB, 1, D), lambda i: (i, 0, 0)),
            pl.BlockSpec((BB, H, S), lambda i: (i, 0, 0)),
            pl.BlockSpec((D, 2 * D), lambda i: (0, 0)),
            pl.BlockSpec((1, D), lambda i: (0, 0)),
        ],
        out_specs=[
            pl.BlockSpec((BB, H, K), lambda i: (i, 0, 0)),
            pl.BlockSpec((BB, H, K), lambda i: (i, 0, 0)),
            pl.BlockSpec((BB, H, K), lambda i: (i, 0, 0)),
            pl.BlockSpec((BB, H, K), lambda i: (i, 0, 0)),
        ],
        out_shape=[
            jax.ShapeDtypeStruct((nb_, H, K), jnp.int32),
            jax.ShapeDtypeStruct((nb_, H, K), jnp.float32),
            jax.ShapeDtypeStruct((nb_, H, K), jnp.int32),
            jax.ShapeDtypeStruct((nb_, H, K), jnp.int32),
        ],
    )(nse, news_repr, user_repr, his_mask, wt, b2)


def _sc_body(per_w, chunk, table_hbm, idx_hbm, w_hbm, out_hbm, idx_v, w_v,
             rows0, rows1, g0, g1, o0, o1):
    wid = lax.axis_index("s") * NC + lax.axis_index("c")
    base = wid * per_w
    icp = pltpu.make_async_copy(idx_hbm.at[pl.ds(base, per_w)], idx_v, g0)
    wcp = pltpu.make_async_copy(w_hbm.at[pl.ds(base, per_w)], w_v, g1)
    icp.start()
    wcp.start()
    icp.wait()
    wcp.wait()

    nch = per_w // chunk
    bufs, gsems, osems = (rows0, rows1), (g0, g1), (o0, o1)
    gathers = [
        pltpu.make_async_copy(
            table_hbm.at[idx_v.at[pl.ds(c * chunk, chunk)]],
            bufs[c % 2], gsems[c % 2])
        for c in range(nch)
    ]
    outs = [
        pltpu.make_async_copy(
            bufs[c % 2], out_hbm.at[pl.ds(base + c * chunk, chunk)],
            osems[c % 2])
        for c in range(nch)
    ]

    def scale(buf, c):
        def group(gi, gcarry):
            w16 = w_v[pl.ds(c * chunk + gi * 16, 16)]
            for i in range(16):
                r = gi * 16 + i
                wvec = jnp.full((16,), w16[i], jnp.float32)
                for j in range(D // 16):
                    sl = pl.ds(j * 16, 16)
                    buf[r, sl] = buf[r, sl] * wvec
            return gcarry
        lax.fori_loop(0, chunk // 16, group, 0)

    gathers[0].start()
    for c in range(nch):
        if c + 1 < nch:
            if c >= 1:
                outs[c - 1].wait()
            gathers[c + 1].start()
        gathers[c].wait()
        scale(bufs[c % 2], c)
        outs[c].start()
    if nch >= 2:
        outs[nch - 2].wait()
    outs[nch - 1].wait()


@functools.lru_cache(maxsize=4)
def _sc_gather_scale(nrows, chunk):
    per_w = nrows // NW
    return pl.kernel(
        functools.partial(_sc_body, per_w, chunk),
        out_type=jax.ShapeDtypeStruct((nrows, D), jnp.float32),
        mesh=plsc.VectorSubcoreMesh(core_axis_name="c", subcore_axis_name="s"),
        scratch_types=[
            pltpu.VMEM((per_w,), jnp.int32),
            pltpu.VMEM((per_w,), jnp.float32),
            pltpu.VMEM((chunk, D), jnp.float32),
            pltpu.VMEM((chunk, D), jnp.float32),
            pltpu.SemaphoreType.DMA,
            pltpu.SemaphoreType.DMA,
            pltpu.SemaphoreType.DMA,
            pltpu.SemaphoreType.DMA,
        ],
    )


def kernel(news_selection_embedding, news_embedding, user_repr, news_repr,
           his_attn_mask, W, b):
    b2 = b.reshape(1, D)
    kid, w, mask, g = _tc_score_topk(
        news_selection_embedding, news_repr, user_repr, his_attn_mask, W, b2)
    table = news_embedding.reshape(B * H * S, D)
    ps_flat = _sc_gather_scale(NROWS, CHUNK)(
        table, g.reshape(NROWS), w.reshape(NROWS))
    return ps_flat.reshape(B, H, K, D), mask, kid
